# baseline (speedup 1.0000x reference)
import functools

import jax
import jax.numpy as jnp
from jax import lax
from jax.experimental import pallas as pl
from jax.experimental.pallas import tpu as pltpu
from jax.experimental.pallas import tpu_sc as plsc

_NC = 2
_NS = 16
_NW = _NC * _NS
_LANES = 16

_B = 1024
_CTX = 20
_D = 64

_B_PER_W = _B // _NW
_IDX_PER_W = _B_PER_W * _CTX
_IDX_CHUNK = 128
_N_CHUNKS = _IDX_PER_W // _IDX_CHUNK


def _context_vectors_sc(x_flat, in_embedding):
    mesh = plsc.VectorSubcoreMesh(core_axis_name="c", subcore_axis_name="s")

    @functools.partial(
        pl.kernel,
        mesh=mesh,
        out_type=jax.ShapeDtypeStruct((_B, _D), jnp.float32),
        compiler_params=pltpu.CompilerParams(use_tc_tiling_on_sc=False),
        scratch_types=[
            pltpu.VMEM((_IDX_PER_W,), jnp.int32),
            pltpu.VMEM((_IDX_PER_W, _D), jnp.float32),
            pltpu.VMEM((_B_PER_W, _D), jnp.float32),
            pltpu.SemaphoreType.DMA,
        ],
    )
    def sc_kernel(x_hbm, emb_hbm, cv_hbm, idx_v, rows_v, cv_v, sem):
        wid = lax.axis_index("s") * _NC + lax.axis_index("c")
        idx_base = wid * _IDX_PER_W
        b_base = wid * _B_PER_W

        pltpu.sync_copy(x_hbm.at[pl.ds(idx_base, _IDX_PER_W)], idx_v)

        copies = []
        for j in range(_N_CHUNKS):
            sl = pl.ds(j * _IDX_CHUNK, _IDX_CHUNK)
            copies.append(
                pltpu.async_copy(emb_hbm.at[idx_v.at[sl]], rows_v.at[sl], sem)
            )
        for cp in copies:
            cp.wait()

        def body(b, _):
            row0 = b * _CTX
            for c in range(_D // _LANES):
                dsl = pl.ds(c * _LANES, _LANES)
                acc = rows_v[row0, dsl]
                for l in range(1, _CTX):
                    acc = acc + rows_v[row0 + l, dsl]
                cv_v[b, dsl] = acc * (1.0 / _CTX)
            return _

        lax.fori_loop(0, _B_PER_W, body, 0)

        pltpu.sync_copy(cv_v, cv_hbm.at[pl.ds(b_base, _B_PER_W)])

    return sc_kernel(x_flat, in_embedding)


def _project_tc(cv, out_weight):
    v = out_weight.shape[0]
    vb = 2048

    def mm_body(cv_ref, w_ref, out_ref):
        out_ref[...] = lax.dot_general(
            cv_ref[...], w_ref[...],
            (((1,), (1,)), ((), ())),
            preferred_element_type=jnp.float32,
        )

    return pl.pallas_call(
        mm_body,
        grid=(pl.cdiv(v, vb),),
        in_specs=[
            pl.BlockSpec((_B, _D), lambda j: (0, 0)),
            pl.BlockSpec((vb, _D), lambda j: (j, 0)),
        ],
        out_specs=pl.BlockSpec((_B, vb), lambda j: (0, j)),
        out_shape=jax.ShapeDtypeStruct((_B, v), jnp.float32),
    )(cv, out_weight)


def kernel(x, in_embedding, out_weight):
    x_flat = jnp.reshape(x.astype(jnp.int32), (-1,))
    cv = _context_vectors_sc(x_flat, in_embedding)
    return _project_tc(cv, out_weight)


# speedup vs baseline: 2.7521x; 2.7521x over previous
import functools

import jax
import jax.numpy as jnp
from jax import lax
from jax.experimental import pallas as pl
from jax.experimental.pallas import tpu as pltpu
from jax.experimental.pallas import tpu_sc as plsc

_NC = 2
_NS = 16
_NW = _NC * _NS
_LANES = 16

_B = 1024
_CTX = 20
_D = 64

_B_PER_W = _B // _NW
_IDX_PER_W = _B_PER_W * _CTX
_IDX_CHUNK = 128
_N_CHUNKS = _IDX_PER_W // _IDX_CHUNK


def _context_vectors_sc(x_flat, in_embedding):
    mesh = plsc.VectorSubcoreMesh(core_axis_name="c", subcore_axis_name="s")

    @functools.partial(
        pl.kernel,
        mesh=mesh,
        out_type=jax.ShapeDtypeStruct((_B, _D), jnp.float32),
        compiler_params=pltpu.CompilerParams(use_tc_tiling_on_sc=False),
        scratch_types=[
            pltpu.VMEM((_IDX_PER_W,), jnp.int32),
            pltpu.VMEM((_IDX_PER_W, _D), jnp.float32),
            pltpu.VMEM((_B_PER_W, _D), jnp.float32),
            pltpu.SemaphoreType.DMA,
        ],
    )
    def sc_kernel(x_hbm, emb_hbm, cv_hbm, idx_v, rows_v, cv_v, sem):
        wid = lax.axis_index("s") * _NC + lax.axis_index("c")
        idx_base = wid * _IDX_PER_W
        b_base = wid * _B_PER_W

        pltpu.sync_copy(x_hbm.at[pl.ds(idx_base, _IDX_PER_W)], idx_v)

        copies = []
        for j in range(_N_CHUNKS):
            sl = pl.ds(j * _IDX_CHUNK, _IDX_CHUNK)
            copies.append(
                pltpu.async_copy(emb_hbm.at[idx_v.at[sl]], rows_v.at[sl], sem)
            )
        for cp in copies:
            cp.wait()

        def body(b, _):
            row0 = b * _CTX
            for c in range(_D // _LANES):
                dsl = pl.ds(c * _LANES, _LANES)
                acc = rows_v[row0, dsl]
                for l in range(1, _CTX):
                    acc = acc + rows_v[row0 + l, dsl]
                cv_v[b, dsl] = acc * (1.0 / _CTX)
            return _

        lax.fori_loop(0, _B_PER_W, body, 0)

        pltpu.sync_copy(cv_v, cv_hbm.at[pl.ds(b_base, _B_PER_W)])

    return sc_kernel(x_flat, in_embedding)


def _project_tc(cv, w_t):
    v = w_t.shape[1]
    vb = 2048

    def mm_body(cv_ref, w_ref, out_ref):
        out_ref[...] = lax.dot_general(
            w_ref[...], cv_ref[...],
            (((0,), (1,)), ((), ())),
            preferred_element_type=jnp.float32,
        )

    return pl.pallas_call(
        mm_body,
        grid=(pl.cdiv(v, vb),),
        in_specs=[
            pl.BlockSpec((_B, _D), lambda j: (0, 0)),
            pl.BlockSpec((_D, vb), lambda j: (0, j)),
        ],
        out_specs=pl.BlockSpec((vb, _B), lambda j: (j, 0)),
        out_shape=jax.ShapeDtypeStruct((v, _B), jnp.float32),
    )(cv, w_t)


def kernel(x, in_embedding, out_weight):
    x_flat = jnp.reshape(x.astype(jnp.int32), (-1,))
    cv = _context_vectors_sc(x_flat, in_embedding)
    return _project_tc(cv, out_weight.T).T
